# baseline (speedup 1.0000x reference)
import functools

import jax
import jax.numpy as jnp
from jax import lax
from jax.experimental import pallas as pl
from jax.experimental.pallas import tpu as pltpu
from jax.experimental.pallas import tpu_sc as plsc

F32 = jnp.float32
NC = 2
NS = 16
GP = 8
RG = 512
RN = 1024
EB = 256
CB = EB // 128
ZB = 128
PADT = 128


def _edge_mlp_body(nsets, nlayers, x_ref, *refs):
    out_refs = refs[-nsets:]
    wrefs = refs[:len(refs) - nsets]
    x = x_ref[...]
    k = 0
    for si in range(nsets):
        h = x
        nl = nlayers[si]
        for li in range(nl):
            w = wrefs[k][...]
            b = wrefs[k + 1][...]
            k += 2
            h = jnp.dot(h.astype(jnp.bfloat16), w,
                        preferred_element_type=F32) + b
            if li < nl - 1:
                h = jnp.maximum(h, 0.0)
        out_refs[si][...] = h


def _edge_mlp(xp, sets):
    rp, kp = xp.shape
    nblk = pl.cdiv(rp, RG)
    in_specs = [pl.BlockSpec((RG, kp), lambda i: (i, 0))]
    args = [xp]
    for layers in sets:
        for w, b in layers:
            in_specs.append(pl.BlockSpec(w.shape, lambda i: (0, 0)))
            in_specs.append(pl.BlockSpec(b.shape, lambda i: (0, 0)))
            args += [w, b]
    out_shape = []
    out_specs = []
    for layers in sets:
        oc = layers[-1][0].shape[1]
        out_shape.append(jax.ShapeDtypeStruct((rp, oc), F32))
        out_specs.append(pl.BlockSpec((RG, oc), lambda i: (i, 0)))
    body = functools.partial(_edge_mlp_body, len(sets), [len(s) for s in sets])
    return pl.pallas_call(body, grid=(nblk,), in_specs=in_specs,
                          out_shape=out_shape, out_specs=out_specs)(*args)


def _node_mlp_body(nlayers, norm, has_add, *refs):
    nin = 1 + 2 * nlayers + (1 if has_add else 0)
    refs, o_ref = refs[:nin], refs[nin]
    if norm:
        a = refs[0][0]
        inv = 1.0 / jnp.maximum(a[:, 24:25], 1.0)
        h = a * inv
    else:
        h = refs[0][...]
    k = 1
    for li in range(nlayers):
        w = refs[k][...]
        b = refs[k + 1][...]
        k += 2
        h = jnp.dot(h.astype(jnp.bfloat16), w,
                    preferred_element_type=F32) + b
        if li < nlayers - 1:
            h = jnp.maximum(h, 0.0)
    if has_add:
        h = h + refs[k][...]
    o_ref[...] = h


def _node_mlp(inp, layers, addend=None):
    oc = layers[-1][0].shape[1]
    norm = inp.ndim == 3
    if norm:
        half = inp.shape[1]
        nb = half // RN
        grid = (2, nb)
        in_specs = [pl.BlockSpec((1, RN, 32), lambda c, i: (c, i, 0))]
        omap = lambda c, i, _nb=nb: (c * _nb + i, 0)
        wmap = lambda c, i: (0, 0)
        ntot = 2 * half
    else:
        ntot, din = inp.shape
        grid = (ntot // RN,)
        in_specs = [pl.BlockSpec((RN, din), lambda i: (i, 0))]
        omap = lambda i: (i, 0)
        wmap = lambda i: (0, 0)
    args = [inp]
    for w, b in layers:
        in_specs.append(pl.BlockSpec(w.shape, wmap))
        in_specs.append(pl.BlockSpec(b.shape, wmap))
        args += [w, b]
    if addend is not None:
        in_specs.append(pl.BlockSpec((RN, oc), omap))
        args.append(addend)
    body = functools.partial(_node_mlp_body, len(layers), norm,
                             addend is not None)
    return pl.pallas_call(body, grid=grid, in_specs=in_specs,
                          out_shape=jax.ShapeDtypeStruct((ntot, oc), F32),
                          out_specs=pl.BlockSpec((RN, oc), omap))(*args)


def _tr_body(x_ref, o_ref):
    o_ref[...] = x_ref[...].T


def _to_rowmajor(a_t):
    k, n = a_t.shape
    tb = 32768
    return pl.pallas_call(
        _tr_body, grid=(pl.cdiv(n, tb),),
        in_specs=[pl.BlockSpec((k, tb), lambda i: (0, i))],
        out_shape=jax.ShapeDtypeStruct((n, k), F32),
        out_specs=pl.BlockSpec((tb, k), lambda i: (i, 0)))(a_t)


def _kron(w, g):
    return jnp.kron(jnp.eye(g, dtype=F32), w)


def _pack_layers(layers, in_dim, out_pad, g, count_col=False):
    packed = []
    n = len(layers)
    for i, (w, b) in enumerate(layers):
        if i == 0 and w.shape[0] != in_dim:
            w = jnp.zeros((in_dim, w.shape[1]), F32).at[:w.shape[0]].set(w)
        if i == n - 1:
            w = jnp.zeros((w.shape[0], out_pad), F32).at[:, :w.shape[1]].set(w)
            b = jnp.zeros((out_pad,), F32).at[:b.shape[0]].set(b)
            if count_col:
                b = b.at[24].set(1.0)
        if g > 1:
            w = _kron(w, g)
            b = jnp.tile(b, g)
        packed.append((w.astype(jnp.bfloat16), b[None, :]))
    return packed


def _mesh():
    return plsc.VectorSubcoreMesh(core_axis_name="c", subcore_axis_name="s")


_SC_PARAMS = pltpu.CompilerParams(use_tc_tiling_on_sc=False,
                                  needs_layout_passes=False)


def _zero_fill(zbuf, agg_sh, s, zr):
    off = 0
    rem = zr
    while rem > 0:
        n = min(rem, ZB)
        pltpu.sync_copy(zbuf.at[pl.ds(0, n)], agg_sh.at[pl.ds(s * zr + off, n)])
        off += n
        rem -= n


def _idx_compute(dstf_v, idx2_v, base, half):
    for j in range(CB):
        for k2 in range(8):
            d = dstf_v[pl.ds((j * 8 + k2) * 16, 16)]
            loc = d - base
            ok = (loc >= 0) & (loc < half)
            idx2_v[j, 0, pl.ds(k2 * 16, 16)] = jnp.where(ok, loc, half)


def _sc_layer(nf, efp, ei_flat, zeros32, half, e_pad):
    eps = e_pad // NS
    nch = eps // EB
    assert nch % 2 == 0
    rpw = half // NS
    zr = (half + PADT) // NS
    gpe = EB // GP

    def body(nf_h, ef_h, ei_h, z_h, out_h, agg_sh,
             ei_v0, ei_v1, ef_v0, ef_v1,
             src_v, idx2_v, rows_v, sem0, sem1, sem_g):
        c = lax.axis_index("c")
        s = lax.axis_index("s")
        pltpu.sync_copy(z_h, rows_v.at[pl.ds(0, ZB)])
        _zero_fill(rows_v, agg_sh, s, zr)
        plsc.subcore_barrier()
        base = c * half
        e_base = s * eps
        iota2 = lax.iota(jnp.int32, 16) * 2
        bufs = ((ei_v0, ef_v0, sem0), (ei_v1, ef_v1, sem1))

        def start_loads(ci, buf):
            iv, ev, sem = buf
            e0 = e_base + ci * EB
            pltpu.async_copy(ei_h.at[pl.ds(2 * e0, 2 * EB)], iv, sem)
            pltpu.async_copy(ef_h.at[pl.ds(e0 // GP, gpe)], ev, sem)

        def wait_loads(buf):
            iv, ev, sem = buf
            pltpu.make_async_copy(ei_h.at[pl.ds(0, 2 * EB)], iv, sem).wait()
            pltpu.make_async_copy(ef_h.at[pl.ds(0, gpe)], ev, sem).wait()

        start_loads(0, bufs[0])
        start_loads(1, bufs[1])

        def super_chunk(i2, carry):
            for b in (0, 1):
                ci = i2 * 2 + b
                iv, ev, _ = bufs[b]
                wait_loads(bufs[b])
                for j in range(CB):
                    for k2 in range(8):
                        o = (j * 8 + k2) * 32
                        src_v[j, 0, pl.ds(k2 * 16, 16)] = plsc.load_gather(
                            iv, [iota2 + o])
                cps = [pltpu.async_copy(nf_h.at[src_v.at[j, 0]],
                                        rows_v.at[pl.ds(j * 128, 128)], sem_g)
                       for j in range(CB)]
                for j in range(CB):
                    for k2 in range(8):
                        o = (j * 8 + k2) * 32
                        d = plsc.load_gather(iv, [iota2 + (o + 1)])
                        loc = d - base
                        ok = (loc >= 0) & (loc < half)
                        idx2_v[j, 0, pl.ds(k2 * 16, 16)] = jnp.where(
                            ok, loc, half)
                for cp in cps:
                    cp.wait()
                def mb(g, cc, _ev=ev):
                    for q in range(GP):
                        r = g * GP + q
                        rows_v[r, pl.ds(0, 16)] = (
                            rows_v[r, pl.ds(0, 16)]
                            * _ev[g, pl.ds(q * 32, 16)])
                        rows_v[r, pl.ds(16, 16)] = (
                            rows_v[r, pl.ds(16, 16)]
                            * _ev[g, pl.ds(q * 32 + 16, 16)])
                    return cc
                lax.fori_loop(0, gpe, mb, 0)
                for j in range(CB):
                    pltpu.sync_copy(rows_v.at[pl.ds(j * 128, 128)],
                                    agg_sh.at[idx2_v.at[j, 0]], add=True)

                @pl.when(ci + 2 < nch)
                def _():
                    start_loads(ci + 2, bufs[b])
            return carry
        lax.fori_loop(0, nch // 2, super_chunk, 0)
        plsc.subcore_barrier()
        pltpu.sync_copy(agg_sh.at[pl.ds(s * rpw, rpw)],
                        out_h.at[c, pl.ds(s * rpw, rpw)])

    return pl.kernel(
        body,
        out_type=jax.ShapeDtypeStruct((2, half, 32), F32),
        mesh=_mesh(),
        compiler_params=_SC_PARAMS,
        scratch_types=[
            pltpu.VMEM_SHARED((half + PADT, 32), F32),
            pltpu.VMEM((2 * EB,), jnp.int32),
            pltpu.VMEM((2 * EB,), jnp.int32),
            pltpu.VMEM((EB // GP, 32 * GP), F32),
            pltpu.VMEM((EB // GP, 32 * GP), F32),
            pltpu.VMEM((CB, 1, 128), jnp.int32),
            pltpu.VMEM((CB, 1, 128), jnp.int32),
            pltpu.VMEM((EB, 32), F32),
            pltpu.SemaphoreType.DMA,
            pltpu.SemaphoreType.DMA,
            pltpu.SemaphoreType.DMA,
        ])(nf, efp, ei_flat, zeros32)


def _sc_gather(table, cl2, npad):
    rw = npad // (NC * NS)
    cw = rw // 128

    def body(t_h, cl_h, out_h, idx_v, rows_v, sem):
        c = lax.axis_index("c")
        s = lax.axis_index("s")
        w = s * NC + c
        pltpu.sync_copy(cl_h.at[pl.ds(w * cw, cw)], idx_v)
        cps = [pltpu.async_copy(t_h.at[idx_v.at[j, 0]],
                                rows_v.at[pl.ds(j * 128, 128)], sem)
               for j in range(cw)]
        for cp in cps:
            cp.wait()
        pltpu.sync_copy(rows_v, out_h.at[pl.ds(w * rw, rw)])

    return pl.kernel(
        body,
        out_type=jax.ShapeDtypeStruct((npad, 8), F32),
        mesh=_mesh(),
        compiler_params=_SC_PARAMS,
        scratch_types=[
            pltpu.VMEM((cw, 1, 128), jnp.int32),
            pltpu.VMEM((rw, 8), F32),
            pltpu.SemaphoreType.DMA,
        ])(table, cl2)


def _branch(x_rm, edge_index, ea_rm, plist, half, aux):
    ntot = 2 * half
    e = ea_rm.shape[0]
    e_pad = NS * EB * 2 * (-(-e // (NS * EB * 2)))
    pad_pair = jnp.tile(jnp.array([0, ntot], jnp.int32), e_pad - e)
    ei_flat = jnp.concatenate([edge_index.T.reshape(2 * e), pad_pair])
    ea = jnp.pad(ea_rm, ((0, e_pad - e), (0, 0)))

    efs = _edge_mlp(ea.reshape(e_pad // GP, 3 * GP),
                    [_pack_layers(p["edge"], 3, 32, GP, count_col=True)
                     for p in plist])

    agg3 = None
    for li, p in enumerate(plist):
        if li == 0:
            nf = _node_mlp(x_rm,
                           _pack_layers(p["density"], 3, 32, 1, count_col=True))
        else:
            nf = _node_mlp(agg3,
                           _pack_layers(p["density"], 32, 32, 1,
                                        count_col=True))
        agg3 = _sc_layer(nf, efs[li], ei_flat, aux["zeros32"], half, e_pad)
    return agg3


def kernel(x, edge_index, edge_attr, x_c, edge_index_c, edge_attr_c, cluster,
           params):
    n_fine = x.shape[0]
    n_coarse = x_c.shape[0]
    half_f = RN * (-(-n_fine // (2 * RN)))
    half_c = RN * (-(-n_coarse // (2 * RN)))
    aux = {"zeros32": jnp.zeros((ZB, 32), F32)}

    ea_rm = _to_rowmajor(edge_attr.T)
    ea_c_rm = _to_rowmajor(edge_attr_c.T)
    x_rm = jnp.pad(_to_rowmajor(x.T), ((0, 2 * half_f - n_fine), (0, 0)))
    x_c_rm = jnp.pad(_to_rowmajor(x_c.T), ((0, 2 * half_c - n_coarse), (0, 0)))

    agg_c = _branch(x_c_rm, edge_index_c, ea_c_rm, params["coarse"],
                    half_c, aux)
    outc8 = _node_mlp(agg_c, _pack_layers(params["readout_coarse"], 32, 8, 1))

    npad_cl = NC * NS * EB * (-(-(2 * half_f) // (NC * NS * EB)))
    cl2 = jnp.pad(cluster, (0, npad_cl - n_fine)).reshape(npad_cl // 128, 1, 128)
    g = _sc_gather(outc8, cl2, npad_cl)

    agg_f = _branch(x_rm, edge_index, ea_rm, params["fine"], half_f, aux)
    out8 = _node_mlp(agg_f, _pack_layers(params["readout_fine"], 32, 8, 1),
                     addend=g)

    return (out8[:n_fine, :3], outc8[:n_coarse, :3])


# speedup vs baseline: 1.1750x; 1.1750x over previous
import functools

import jax
import jax.numpy as jnp
from jax import lax
from jax.experimental import pallas as pl
from jax.experimental.pallas import tpu as pltpu
from jax.experimental.pallas import tpu_sc as plsc

F32 = jnp.float32
NC = 2
NS = 16
GP = 8
RG = 512
RN = 1024
EB = 256
CB = EB // 128
ZB = 128
PADT = 128


def _edge_mlp_body(nsets, nlayers, x_ref, *refs):
    out_refs = refs[-nsets:]
    wrefs = refs[:len(refs) - nsets]
    x = x_ref[...]
    k = 0
    for si in range(nsets):
        h = x
        nl = nlayers[si]
        for li in range(nl):
            w = wrefs[k][...]
            b = wrefs[k + 1][...]
            k += 2
            h = jnp.dot(h.astype(jnp.bfloat16), w,
                        preferred_element_type=F32) + b
            if li < nl - 1:
                h = jnp.maximum(h, 0.0)
        out_refs[si][...] = h


def _edge_mlp(xp, sets):
    rp, kp = xp.shape
    nblk = pl.cdiv(rp, RG)
    in_specs = [pl.BlockSpec((RG, kp), lambda i: (i, 0))]
    args = [xp]
    for layers in sets:
        for w, b in layers:
            in_specs.append(pl.BlockSpec(w.shape, lambda i: (0, 0)))
            in_specs.append(pl.BlockSpec(b.shape, lambda i: (0, 0)))
            args += [w, b]
    out_shape = []
    out_specs = []
    for layers in sets:
        oc = layers[-1][0].shape[1]
        out_shape.append(jax.ShapeDtypeStruct((rp, oc), F32))
        out_specs.append(pl.BlockSpec((RG, oc), lambda i: (i, 0)))
    body = functools.partial(_edge_mlp_body, len(sets), [len(s) for s in sets])
    return pl.pallas_call(body, grid=(nblk,), in_specs=in_specs,
                          out_shape=out_shape, out_specs=out_specs)(*args)


def _node_mlp_body(nlayers, norm, has_add, *refs):
    nin = 1 + 2 * nlayers + (1 if has_add else 0)
    refs, o_ref = refs[:nin], refs[nin]
    if norm:
        a = refs[0][0]
        inv = 1.0 / jnp.maximum(a[:, 24:25], 1.0)
        h = a * inv
    else:
        h = refs[0][...]
    k = 1
    for li in range(nlayers):
        w = refs[k][...]
        b = refs[k + 1][...]
        k += 2
        h = jnp.dot(h.astype(jnp.bfloat16), w,
                    preferred_element_type=F32) + b
        if li < nlayers - 1:
            h = jnp.maximum(h, 0.0)
    if has_add:
        h = h + refs[k][...]
    o_ref[...] = h


def _node_mlp(inp, layers, addend=None):
    oc = layers[-1][0].shape[1]
    norm = inp.ndim == 3
    if norm:
        half = inp.shape[1]
        nb = half // RN
        grid = (2, nb)
        in_specs = [pl.BlockSpec((1, RN, 32), lambda c, i: (c, i, 0))]
        omap = lambda c, i, _nb=nb: (c * _nb + i, 0)
        wmap = lambda c, i: (0, 0)
        ntot = 2 * half
    else:
        ntot, din = inp.shape
        grid = (ntot // RN,)
        in_specs = [pl.BlockSpec((RN, din), lambda i: (i, 0))]
        omap = lambda i: (i, 0)
        wmap = lambda i: (0, 0)
    args = [inp]
    for w, b in layers:
        in_specs.append(pl.BlockSpec(w.shape, wmap))
        in_specs.append(pl.BlockSpec(b.shape, wmap))
        args += [w, b]
    if addend is not None:
        in_specs.append(pl.BlockSpec((RN, oc), omap))
        args.append(addend)
    body = functools.partial(_node_mlp_body, len(layers), norm,
                             addend is not None)
    return pl.pallas_call(body, grid=grid, in_specs=in_specs,
                          out_shape=jax.ShapeDtypeStruct((ntot, oc), F32),
                          out_specs=pl.BlockSpec((RN, oc), omap))(*args)


def _tr_body(x_ref, o_ref):
    o_ref[...] = x_ref[...].T


def _to_rowmajor(a_t):
    k, n = a_t.shape
    tb = 32768
    return pl.pallas_call(
        _tr_body, grid=(pl.cdiv(n, tb),),
        in_specs=[pl.BlockSpec((k, tb), lambda i: (0, i))],
        out_shape=jax.ShapeDtypeStruct((n, k), F32),
        out_specs=pl.BlockSpec((tb, k), lambda i: (i, 0)))(a_t)


def _kron(w, g):
    return jnp.kron(jnp.eye(g, dtype=F32), w)


def _pack_layers(layers, in_dim, out_pad, g, count_col=False):
    packed = []
    n = len(layers)
    for i, (w, b) in enumerate(layers):
        if i == 0 and w.shape[0] != in_dim:
            w = jnp.zeros((in_dim, w.shape[1]), F32).at[:w.shape[0]].set(w)
        if i == n - 1:
            w = jnp.zeros((w.shape[0], out_pad), F32).at[:, :w.shape[1]].set(w)
            b = jnp.zeros((out_pad,), F32).at[:b.shape[0]].set(b)
            if count_col:
                b = b.at[24].set(1.0)
        if g > 1:
            w = _kron(w, g)
            b = jnp.tile(b, g)
        packed.append((w.astype(jnp.bfloat16), b[None, :]))
    return packed


def _mesh():
    return plsc.VectorSubcoreMesh(core_axis_name="c", subcore_axis_name="s")


_SC_PARAMS = pltpu.CompilerParams(use_tc_tiling_on_sc=False)


def _zero_fill(zbuf, agg_sh, s, zr):
    off = 0
    rem = zr
    while rem > 0:
        n = min(rem, ZB)
        pltpu.sync_copy(zbuf.at[pl.ds(0, n)], agg_sh.at[pl.ds(s * zr + off, n)])
        off += n
        rem -= n


def _idx_compute(dstf_v, idx2_v, base, half):
    for j in range(CB):
        for k2 in range(8):
            d = dstf_v[pl.ds((j * 8 + k2) * 16, 16)]
            loc = d - base
            ok = (loc >= 0) & (loc < half)
            idx2_v[j, 0, pl.ds(k2 * 16, 16)] = jnp.where(ok, loc, half)


def _sc_layer(nf, efp, src2, dstf, zeros32, half, e_pad):
    eps = e_pad // NS
    nch = eps // EB
    assert nch % 2 == 0
    rpw = half // NS
    zr = (half + PADT) // NS
    gpe = EB // GP

    def body(nf_h, ef_h, src_h, dst_h, z_h, out_h, agg_sh,
             src_v0, src_v1, dstf_v0, dstf_v1, ef_v0, ef_v1,
             idx2_v, rows_v, sem0, sem1, sem_g):
        c = lax.axis_index("c")
        s = lax.axis_index("s")
        pltpu.sync_copy(z_h, rows_v.at[pl.ds(0, ZB)])
        _zero_fill(rows_v, agg_sh, s, zr)
        plsc.subcore_barrier()
        base = c * half
        e_base = s * eps
        bufs = ((src_v0, dstf_v0, ef_v0, sem0), (src_v1, dstf_v1, ef_v1, sem1))

        def start_loads(ci, buf):
            sv, dv, ev, sem = buf
            e0 = e_base + ci * EB
            pltpu.async_copy(src_h.at[pl.ds(e0 // 128, CB)], sv, sem)
            pltpu.async_copy(dst_h.at[pl.ds(e0, EB)], dv, sem)
            pltpu.async_copy(ef_h.at[pl.ds(e0 // GP, gpe)], ev, sem)

        def wait_loads(buf):
            sv, dv, ev, sem = buf
            pltpu.make_async_copy(src_h.at[pl.ds(0, CB)], sv, sem).wait()
            pltpu.make_async_copy(dst_h.at[pl.ds(0, EB)], dv, sem).wait()
            pltpu.make_async_copy(ef_h.at[pl.ds(0, gpe)], ev, sem).wait()

        start_loads(0, bufs[0])
        start_loads(1, bufs[1])

        def super_chunk(i2, carry):
            for b in (0, 1):
                ci = i2 * 2 + b
                buf = bufs[b]
                sv, dv, ev, _ = buf
                wait_loads(buf)
                cps = [pltpu.async_copy(nf_h.at[sv.at[j, 0]],
                                        rows_v.at[pl.ds(j * 128, 128)], sem_g)
                       for j in range(CB)]
                _idx_compute(dv, idx2_v, base, half)
                for cp in cps:
                    cp.wait()

                def mb(g, cc, _ev=ev):
                    for q in range(GP):
                        r = g * GP + q
                        rows_v[r, pl.ds(0, 16)] = (
                            rows_v[r, pl.ds(0, 16)]
                            * _ev[g, pl.ds(q * 32, 16)])
                        rows_v[r, pl.ds(16, 16)] = (
                            rows_v[r, pl.ds(16, 16)]
                            * _ev[g, pl.ds(q * 32 + 16, 16)])
                    return cc
                lax.fori_loop(0, gpe, mb, 0)
                for j in range(CB):
                    pltpu.sync_copy(rows_v.at[pl.ds(j * 128, 128)],
                                    agg_sh.at[idx2_v.at[j, 0]], add=True)

                @pl.when(ci + 2 < nch)
                def _():
                    start_loads(ci + 2, buf)
            return carry
        lax.fori_loop(0, nch // 2, super_chunk, 0)
        plsc.subcore_barrier()
        pltpu.sync_copy(agg_sh.at[pl.ds(s * rpw, rpw)],
                        out_h.at[c, pl.ds(s * rpw, rpw)])

    return pl.kernel(
        body,
        out_type=jax.ShapeDtypeStruct((2, half, 32), F32),
        mesh=_mesh(),
        compiler_params=_SC_PARAMS,
        scratch_types=[
            pltpu.VMEM_SHARED((half + PADT, 32), F32),
            pltpu.VMEM((CB, 1, 128), jnp.int32),
            pltpu.VMEM((CB, 1, 128), jnp.int32),
            pltpu.VMEM((EB,), jnp.int32),
            pltpu.VMEM((EB,), jnp.int32),
            pltpu.VMEM((EB // GP, 32 * GP), F32),
            pltpu.VMEM((EB // GP, 32 * GP), F32),
            pltpu.VMEM((CB, 1, 128), jnp.int32),
            pltpu.VMEM((EB, 32), F32),
            pltpu.SemaphoreType.DMA,
            pltpu.SemaphoreType.DMA,
            pltpu.SemaphoreType.DMA,
        ])(nf, efp, src2, dstf, zeros32)


def _sc_gather(table, cl2, npad):
    rw = npad // (NC * NS)
    cw = rw // 128

    def body(t_h, cl_h, out_h, idx_v, rows_v, sem):
        c = lax.axis_index("c")
        s = lax.axis_index("s")
        w = s * NC + c
        pltpu.sync_copy(cl_h.at[pl.ds(w * cw, cw)], idx_v)
        cps = [pltpu.async_copy(t_h.at[idx_v.at[j, 0]],
                                rows_v.at[pl.ds(j * 128, 128)], sem)
               for j in range(cw)]
        for cp in cps:
            cp.wait()
        pltpu.sync_copy(rows_v, out_h.at[pl.ds(w * rw, rw)])

    return pl.kernel(
        body,
        out_type=jax.ShapeDtypeStruct((npad, 8), F32),
        mesh=_mesh(),
        compiler_params=_SC_PARAMS,
        scratch_types=[
            pltpu.VMEM((cw, 1, 128), jnp.int32),
            pltpu.VMEM((rw, 8), F32),
            pltpu.SemaphoreType.DMA,
        ])(table, cl2)


def _branch(x_rm, edge_index, ea_rm, plist, half, aux):
    ntot = 2 * half
    e = ea_rm.shape[0]
    e_pad = NS * EB * 2 * (-(-e // (NS * EB * 2)))
    src = jnp.pad(edge_index[0], (0, e_pad - e))
    dst = jnp.pad(edge_index[1], (0, e_pad - e), constant_values=ntot)
    src2 = src.reshape(e_pad // 128, 1, 128)
    ea = jnp.pad(ea_rm, ((0, e_pad - e), (0, 0)))

    efs = _edge_mlp(ea.reshape(e_pad // GP, 3 * GP),
                    [_pack_layers(p["edge"], 3, 32, GP, count_col=True)
                     for p in plist])

    agg3 = None
    for li, p in enumerate(plist):
        if li == 0:
            nf = _node_mlp(x_rm,
                           _pack_layers(p["density"], 3, 32, 1, count_col=True))
        else:
            nf = _node_mlp(agg3,
                           _pack_layers(p["density"], 32, 32, 1,
                                        count_col=True))
        agg3 = _sc_layer(nf, efs[li], src2, dst, aux["zeros32"], half, e_pad)
    return agg3


def kernel(x, edge_index, edge_attr, x_c, edge_index_c, edge_attr_c, cluster,
           params):
    n_fine = x.shape[0]
    n_coarse = x_c.shape[0]
    half_f = RN * (-(-n_fine // (2 * RN)))
    half_c = RN * (-(-n_coarse // (2 * RN)))
    aux = {"zeros32": jnp.zeros((ZB, 32), F32)}

    ea_rm = _to_rowmajor(edge_attr.T)
    ea_c_rm = _to_rowmajor(edge_attr_c.T)
    x_rm = jnp.pad(_to_rowmajor(x.T), ((0, 2 * half_f - n_fine), (0, 0)))
    x_c_rm = jnp.pad(_to_rowmajor(x_c.T), ((0, 2 * half_c - n_coarse), (0, 0)))

    agg_c = _branch(x_c_rm, edge_index_c, ea_c_rm, params["coarse"],
                    half_c, aux)
    outc8 = _node_mlp(agg_c, _pack_layers(params["readout_coarse"], 32, 8, 1))

    npad_cl = NC * NS * EB * (-(-(2 * half_f) // (NC * NS * EB)))
    cl2 = jnp.pad(cluster, (0, npad_cl - n_fine)).reshape(npad_cl // 128, 1, 128)
    g = _sc_gather(outc8, cl2, npad_cl)

    agg_f = _branch(x_rm, edge_index, ea_rm, params["fine"], half_f, aux)
    out8 = _node_mlp(agg_f, _pack_layers(params["readout_fine"], 32, 8, 1),
                     addend=g)

    return (out8[:n_fine, :3], outc8[:n_coarse, :3])


# speedup vs baseline: 1.1982x; 1.0197x over previous
import functools

import jax
import jax.numpy as jnp
from jax import lax
from jax.experimental import pallas as pl
from jax.experimental.pallas import tpu as pltpu
from jax.experimental.pallas import tpu_sc as plsc

F32 = jnp.float32
NC = 2
NS = 16
GP = 8
RG = 512
RN = 1024
EB = 256
CB = EB // 128
ZB = 128
PADT = 128


def _edge_mlp_body(nsets, nlayers, x_ref, *refs):
    out_refs = refs[-nsets:]
    wrefs = refs[:len(refs) - nsets]
    x = x_ref[...]
    k = 0
    for si in range(nsets):
        h = x
        nl = nlayers[si]
        for li in range(nl):
            w = wrefs[k][...]
            b = wrefs[k + 1][...]
            k += 2
            h = jnp.dot(h.astype(jnp.bfloat16), w,
                        preferred_element_type=F32) + b
            if li < nl - 1:
                h = jnp.maximum(h, 0.0)
        out_refs[si][...] = h


def _edge_mlp(xp, sets):
    rp, kp = xp.shape
    nblk = pl.cdiv(rp, RG)
    in_specs = [pl.BlockSpec((RG, kp), lambda i: (i, 0))]
    args = [xp]
    for layers in sets:
        for w, b in layers:
            in_specs.append(pl.BlockSpec(w.shape, lambda i: (0, 0)))
            in_specs.append(pl.BlockSpec(b.shape, lambda i: (0, 0)))
            args += [w, b]
    out_shape = []
    out_specs = []
    for layers in sets:
        oc = layers[-1][0].shape[1]
        out_shape.append(jax.ShapeDtypeStruct((rp, oc), F32))
        out_specs.append(pl.BlockSpec((RG, oc), lambda i: (i, 0)))
    body = functools.partial(_edge_mlp_body, len(sets), [len(s) for s in sets])
    return pl.pallas_call(body, grid=(nblk,), in_specs=in_specs,
                          out_shape=out_shape, out_specs=out_specs)(*args)


def _node_mlp_body(nlayers, norm, has_add, *refs):
    nin = 1 + 2 * nlayers + (1 if has_add else 0)
    refs, o_ref = refs[:nin], refs[nin]
    if norm:
        a = refs[0][0]
        inv = 1.0 / jnp.maximum(a[:, 24:25], 1.0)
        h = a * inv
    else:
        h = refs[0][...]
    k = 1
    for li in range(nlayers):
        w = refs[k][...]
        b = refs[k + 1][...]
        k += 2
        h = jnp.dot(h.astype(jnp.bfloat16), w,
                    preferred_element_type=F32) + b
        if li < nlayers - 1:
            h = jnp.maximum(h, 0.0)
    if has_add:
        h = h + refs[k][...]
    o_ref[...] = h


def _node_mlp(inp, layers, addend=None):
    oc = layers[-1][0].shape[1]
    norm = inp.ndim == 3
    if norm:
        half = inp.shape[1]
        nb = half // RN
        grid = (2, nb)
        in_specs = [pl.BlockSpec((1, RN, 32), lambda c, i: (c, i, 0))]
        omap = lambda c, i, _nb=nb: (c * _nb + i, 0)
        wmap = lambda c, i: (0, 0)
        ntot = 2 * half
    else:
        ntot, din = inp.shape
        grid = (ntot // RN,)
        in_specs = [pl.BlockSpec((RN, din), lambda i: (i, 0))]
        omap = lambda i: (i, 0)
        wmap = lambda i: (0, 0)
    args = [inp]
    for w, b in layers:
        in_specs.append(pl.BlockSpec(w.shape, wmap))
        in_specs.append(pl.BlockSpec(b.shape, wmap))
        args += [w, b]
    if addend is not None:
        in_specs.append(pl.BlockSpec((RN, oc), omap))
        args.append(addend)
    body = functools.partial(_node_mlp_body, len(layers), norm,
                             addend is not None)
    return pl.pallas_call(body, grid=grid, in_specs=in_specs,
                          out_shape=jax.ShapeDtypeStruct((ntot, oc), F32),
                          out_specs=pl.BlockSpec((RN, oc), omap))(*args)


def _tr_body(x_ref, o_ref):
    o_ref[...] = x_ref[...].T


def _to_rowmajor(a_t):
    k, n = a_t.shape
    tb = 32768
    return pl.pallas_call(
        _tr_body, grid=(pl.cdiv(n, tb),),
        in_specs=[pl.BlockSpec((k, tb), lambda i: (0, i))],
        out_shape=jax.ShapeDtypeStruct((n, k), F32),
        out_specs=pl.BlockSpec((tb, k), lambda i: (i, 0)))(a_t)


def _kron(w, g):
    return jnp.kron(jnp.eye(g, dtype=F32), w)


def _pack_layers(layers, in_dim, out_pad, g, count_col=False):
    packed = []
    n = len(layers)
    for i, (w, b) in enumerate(layers):
        if i == 0 and w.shape[0] != in_dim:
            w = jnp.zeros((in_dim, w.shape[1]), F32).at[:w.shape[0]].set(w)
        if i == n - 1:
            w = jnp.zeros((w.shape[0], out_pad), F32).at[:, :w.shape[1]].set(w)
            b = jnp.zeros((out_pad,), F32).at[:b.shape[0]].set(b)
            if count_col:
                b = b.at[24].set(1.0)
        if g > 1:
            w = _kron(w, g)
            b = jnp.tile(b, g)
        packed.append((w.astype(jnp.bfloat16), b[None, :]))
    return packed


def _mesh():
    return plsc.VectorSubcoreMesh(core_axis_name="c", subcore_axis_name="s")


_SC_PARAMS = pltpu.CompilerParams(use_tc_tiling_on_sc=False)


def _zero_fill(zbuf, agg_sh, s, zr):
    off = 0
    rem = zr
    while rem > 0:
        n = min(rem, ZB)
        pltpu.sync_copy(zbuf.at[pl.ds(0, n)], agg_sh.at[pl.ds(s * zr + off, n)])
        off += n
        rem -= n


def _idx_compute(dstf_v, idx2_v, base, half):
    for j in range(CB):
        for k2 in range(8):
            d = dstf_v[pl.ds((j * 8 + k2) * 16, 16)]
            loc = d - base
            ok = (loc >= 0) & (loc < half)
            idx2_v[j, 0, pl.ds(k2 * 16, 16)] = jnp.where(ok, loc, half)


def _sc_layer(nf, efp, src2, dstf, zeros32, half, e_pad):
    eps = e_pad // NS
    nch = eps // EB
    assert nch % 2 == 0
    rpw = half // NS
    zr = (half + PADT) // NS
    gpe = EB // GP

    def body(nf_h, ef_h, src_h, dst_h, z_h, out_h, agg_sh,
             src_v0, src_v1, dstf_v0, dstf_v1, ef_v0, ef_v1,
             idx2_v, rows_v, sem0, sem1, sem_g, sem_sc):
        c = lax.axis_index("c")
        s = lax.axis_index("s")
        pltpu.sync_copy(z_h, rows_v.at[pl.ds(0, ZB)])
        _zero_fill(rows_v, agg_sh, s, zr)
        plsc.subcore_barrier()
        base = c * half
        e_base = s * eps
        bufs = ((src_v0, dstf_v0, ef_v0, sem0), (src_v1, dstf_v1, ef_v1, sem1))

        def start_loads(ci, buf):
            sv, dv, ev, sem = buf
            e0 = e_base + ci * EB
            pltpu.async_copy(src_h.at[pl.ds(e0 // 128, CB)], sv, sem)
            pltpu.async_copy(dst_h.at[pl.ds(e0, EB)], dv, sem)
            pltpu.async_copy(ef_h.at[pl.ds(e0 // GP, gpe)], ev, sem)

        def wait_loads(buf):
            sv, dv, ev, sem = buf
            pltpu.make_async_copy(src_h.at[pl.ds(0, CB)], sv, sem).wait()
            pltpu.make_async_copy(dst_h.at[pl.ds(0, EB)], dv, sem).wait()
            pltpu.make_async_copy(ef_h.at[pl.ds(0, gpe)], ev, sem).wait()

        start_loads(0, bufs[0])
        start_loads(1, bufs[1])

        def wait_scatters(n):
            for _ in range(n):
                pltpu.make_async_copy(
                    rows_v.at[pl.ds(0, 128)],
                    agg_sh.at[idx2_v.at[0, 0]], sem_sc).wait()

        def super_chunk(i2, carry):
            for b in (0, 1):
                ci = i2 * 2 + b
                buf = bufs[b]
                sv, dv, ev, _ = buf
                wait_loads(buf)

                @pl.when(ci > 0)
                def _():
                    wait_scatters(CB)

                cps = [pltpu.async_copy(nf_h.at[sv.at[j, 0]],
                                        rows_v.at[pl.ds(j * 128, 128)], sem_g)
                       for j in range(CB)]
                _idx_compute(dv, idx2_v, base, half)
                for cp in cps:
                    cp.wait()

                def mb(g, cc, _ev=ev):
                    for q in range(GP):
                        r = g * GP + q
                        rows_v[r, pl.ds(0, 16)] = (
                            rows_v[r, pl.ds(0, 16)]
                            * _ev[g, pl.ds(q * 32, 16)])
                        rows_v[r, pl.ds(16, 16)] = (
                            rows_v[r, pl.ds(16, 16)]
                            * _ev[g, pl.ds(q * 32 + 16, 16)])
                    return cc
                lax.fori_loop(0, gpe, mb, 0)

                @pl.when(ci + 2 < nch)
                def _():
                    start_loads(ci + 2, buf)
                for j in range(CB):
                    pltpu.async_copy(rows_v.at[pl.ds(j * 128, 128)],
                                     agg_sh.at[idx2_v.at[j, 0]], sem_sc,
                                     add=True)
            return carry
        lax.fori_loop(0, nch // 2, super_chunk, 0)
        wait_scatters(CB)
        plsc.subcore_barrier()
        pltpu.sync_copy(agg_sh.at[pl.ds(s * rpw, rpw)],
                        out_h.at[c, pl.ds(s * rpw, rpw)])

    return pl.kernel(
        body,
        out_type=jax.ShapeDtypeStruct((2, half, 32), F32),
        mesh=_mesh(),
        compiler_params=_SC_PARAMS,
        scratch_types=[
            pltpu.VMEM_SHARED((half + PADT, 32), F32),
            pltpu.VMEM((CB, 1, 128), jnp.int32),
            pltpu.VMEM((CB, 1, 128), jnp.int32),
            pltpu.VMEM((EB,), jnp.int32),
            pltpu.VMEM((EB,), jnp.int32),
            pltpu.VMEM((EB // GP, 32 * GP), F32),
            pltpu.VMEM((EB // GP, 32 * GP), F32),
            pltpu.VMEM((CB, 1, 128), jnp.int32),
            pltpu.VMEM((EB, 32), F32),
            pltpu.SemaphoreType.DMA,
            pltpu.SemaphoreType.DMA,
            pltpu.SemaphoreType.DMA,
            pltpu.SemaphoreType.DMA,
        ])(nf, efp, src2, dstf, zeros32)


def _sc_gather(table, cl2, npad):
    rw = npad // (NC * NS)
    cw = rw // 128

    def body(t_h, cl_h, out_h, idx_v, rows_v, sem):
        c = lax.axis_index("c")
        s = lax.axis_index("s")
        w = s * NC + c
        pltpu.sync_copy(cl_h.at[pl.ds(w * cw, cw)], idx_v)
        cps = [pltpu.async_copy(t_h.at[idx_v.at[j, 0]],
                                rows_v.at[pl.ds(j * 128, 128)], sem)
               for j in range(cw)]
        for cp in cps:
            cp.wait()
        pltpu.sync_copy(rows_v, out_h.at[pl.ds(w * rw, rw)])

    return pl.kernel(
        body,
        out_type=jax.ShapeDtypeStruct((npad, 8), F32),
        mesh=_mesh(),
        compiler_params=_SC_PARAMS,
        scratch_types=[
            pltpu.VMEM((cw, 1, 128), jnp.int32),
            pltpu.VMEM((rw, 8), F32),
            pltpu.SemaphoreType.DMA,
        ])(table, cl2)


def _branch(x_rm, edge_index, ea_rm, plist, half, aux):
    ntot = 2 * half
    e = ea_rm.shape[0]
    e_pad = NS * EB * 2 * (-(-e // (NS * EB * 2)))
    src = jnp.pad(edge_index[0], (0, e_pad - e))
    dst = jnp.pad(edge_index[1], (0, e_pad - e), constant_values=ntot)
    src2 = src.reshape(e_pad // 128, 1, 128)
    ea = jnp.pad(ea_rm, ((0, e_pad - e), (0, 0)))

    efs = _edge_mlp(ea.reshape(e_pad // GP, 3 * GP),
                    [_pack_layers(p["edge"], 3, 32, GP, count_col=True)
                     for p in plist])

    agg3 = None
    for li, p in enumerate(plist):
        if li == 0:
            nf = _node_mlp(x_rm,
                           _pack_layers(p["density"], 3, 32, 1, count_col=True))
        else:
            nf = _node_mlp(agg3,
                           _pack_layers(p["density"], 32, 32, 1,
                                        count_col=True))
        agg3 = _sc_layer(nf, efs[li], src2, dst, aux["zeros32"], half, e_pad)
    return agg3


def kernel(x, edge_index, edge_attr, x_c, edge_index_c, edge_attr_c, cluster,
           params):
    n_fine = x.shape[0]
    n_coarse = x_c.shape[0]
    half_f = RN * (-(-n_fine // (2 * RN)))
    half_c = RN * (-(-n_coarse // (2 * RN)))
    aux = {"zeros32": jnp.zeros((ZB, 32), F32)}

    ea_rm = _to_rowmajor(edge_attr.T)
    ea_c_rm = _to_rowmajor(edge_attr_c.T)
    x_rm = jnp.pad(_to_rowmajor(x.T), ((0, 2 * half_f - n_fine), (0, 0)))
    x_c_rm = jnp.pad(_to_rowmajor(x_c.T), ((0, 2 * half_c - n_coarse), (0, 0)))

    agg_c = _branch(x_c_rm, edge_index_c, ea_c_rm, params["coarse"],
                    half_c, aux)
    outc8 = _node_mlp(agg_c, _pack_layers(params["readout_coarse"], 32, 8, 1))

    npad_cl = NC * NS * EB * (-(-(2 * half_f) // (NC * NS * EB)))
    cl2 = jnp.pad(cluster, (0, npad_cl - n_fine)).reshape(npad_cl // 128, 1, 128)
    g = _sc_gather(outc8, cl2, npad_cl)

    agg_f = _branch(x_rm, edge_index, ea_rm, params["fine"], half_f, aux)
    out8 = _node_mlp(agg_f, _pack_layers(params["readout_fine"], 32, 8, 1),
                     addend=g)

    return (out8[:n_fine, :3], outc8[:n_coarse, :3])
